# baseline (speedup 1.0000x reference)
import dataclasses
import functools

import jax
import jax.numpy as jnp
from jax import lax
from jax.experimental import pallas as pl
from jax.experimental.pallas import tpu as pltpu
from jax.experimental.pallas import tpu_sc as plsc

EPS = 1e-12


def _fused_sc(tok_table, pos_table, gamma, beta, ids, n_rows, t_len, d):
    info = plsc.get_sparse_core_info()
    nw = info.num_cores * info.num_subcores
    lanes = info.num_lanes
    rows_per_w = n_rows // nw
    chunk = 64
    nvec = d // lanes

    mesh = plsc.VectorSubcoreMesh(core_axis_name="c", subcore_axis_name="s")

    cp = pltpu.CompilerParams()
    if "needs_layout_passes" in pltpu.CompilerParams.__dataclass_fields__:
        cp = dataclasses.replace(cp, needs_layout_passes=False)

    @functools.partial(
        pl.kernel,
        mesh=mesh,
        compiler_params=cp,
        out_type=jax.ShapeDtypeStruct((n_rows, d), jnp.float32),
        scratch_types=[
            pltpu.VMEM((chunk,), jnp.int32),
            pltpu.VMEM((chunk, d), jnp.float32),
            pltpu.VMEM((chunk, d), jnp.float32),
            pltpu.VMEM((1, d), jnp.float32),
            pltpu.VMEM((1, d), jnp.float32),
            pltpu.SemaphoreType.DMA,
        ],
    )
    def fused_kernel(table_hbm, pos_hbm, gamma_hbm, beta_hbm, idx_hbm, out_hbm,
                     idx_v, h_v, pos_v, g_v, b_v, sem):
        wid = lax.axis_index("s") * info.num_cores + lax.axis_index("c")
        base = wid * rows_per_w
        pos_base = base % t_len

        pltpu.sync_copy(gamma_hbm, g_v)
        pltpu.sync_copy(beta_hbm, b_v)

        @pl.loop(0, rows_per_w, step=chunk)
        def _(c):
            pltpu.sync_copy(idx_hbm.at[pl.ds(base + c, chunk)], idx_v)
            cp_tok = pltpu.async_copy(table_hbm.at[idx_v], h_v, sem)
            cp_pos = pltpu.async_copy(
                pos_hbm.at[pl.ds(pos_base + c, chunk)], pos_v, sem)
            cp_tok.wait()
            cp_pos.wait()

            @pl.loop(0, chunk)
            def _(r):
                s = jnp.zeros((lanes,), jnp.float32)
                q = jnp.zeros((lanes,), jnp.float32)
                for j in range(nvec):
                    sl = pl.ds(j * lanes, lanes)
                    h = h_v[r, sl] + pos_v[r, sl]
                    h_v[r, sl] = h
                    s = s + h
                    q = q + h * h
                mu = jnp.sum(s) * (1.0 / d)
                var = jnp.sum(q) * (1.0 / d) - mu * mu
                x = jnp.full((lanes,), var + EPS, jnp.float32)
                i = plsc.bitcast(x, jnp.int32)
                i = jnp.int32(0x5F3759DF) - (i >> 1)
                y = plsc.bitcast(i, jnp.float32)
                for _ in range(3):
                    y = y * (1.5 - 0.5 * x * y * y)
                mu_v = jnp.full((lanes,), mu, jnp.float32)
                for j in range(nvec):
                    sl = pl.ds(j * lanes, lanes)
                    h_v[r, sl] = (h_v[r, sl] - mu_v) * (y * g_v[0, sl]) + b_v[0, sl]

            pltpu.sync_copy(h_v, out_hbm.at[pl.ds(base + c, chunk)])

    return fused_kernel(tok_table, pos_table, gamma, beta, ids)


def kernel(input_ids, tok_table, pos_table, gamma, beta):
    b, t = input_ids.shape
    v, d = tok_table.shape
    n_rows = b * t

    ids = input_ids.reshape(-1).astype(jnp.int32)
    out = _fused_sc(
        tok_table,
        pos_table,
        gamma.reshape(1, d),
        beta.reshape(1, d),
        ids,
        n_rows,
        t,
        d,
    )
    return out.reshape(b, t, d)


# speedup vs baseline: 3.3518x; 3.3518x over previous
import functools

import jax
import jax.numpy as jnp
from jax import lax
from jax.experimental import pallas as pl
from jax.experimental.pallas import tpu as pltpu
from jax.experimental.pallas import tpu_sc as plsc

EPS = 1e-12


def _sc_gather(tok_table, ids, n_rows, d):
    info = plsc.get_sparse_core_info()
    nw = info.num_cores * info.num_subcores
    rows_per_w = n_rows // nw
    chunk = 128

    mesh = plsc.VectorSubcoreMesh(core_axis_name="c", subcore_axis_name="s")

    @functools.partial(
        pl.kernel,
        mesh=mesh,
        out_type=jax.ShapeDtypeStruct((n_rows, d), jnp.float32),
        scratch_types=[
            pltpu.VMEM((chunk,), jnp.int32),
            pltpu.VMEM((chunk, d), jnp.float32),
            pltpu.SemaphoreType.DMA,
        ],
    )
    def gather_kernel(table_hbm, idx_hbm, out_hbm, idx_v, rows_v, sem):
        wid = lax.axis_index("s") * info.num_cores + lax.axis_index("c")
        base = wid * rows_per_w

        @pl.loop(0, rows_per_w, step=chunk)
        def _(c):
            pltpu.sync_copy(idx_hbm.at[pl.ds(base + c, chunk)], idx_v)
            pltpu.async_copy(table_hbm.at[idx_v], rows_v, sem).wait()
            pltpu.sync_copy(rows_v, out_hbm.at[pl.ds(base + c, chunk)])

    return gather_kernel(tok_table, ids)


def _tc_add_ln(gathered, pos_table, gamma2, beta2, n_rows, t_len, d, blk):
    pos_blocks = t_len // blk

    def body(g_ref, p_ref, gm_ref, bt_ref, o_ref):
        i = pl.program_id(0)
        h = g_ref[...] + p_ref[pl.ds((i % pos_blocks) * blk, blk), :]
        mu = jnp.mean(h, axis=1, keepdims=True)
        hc = h - mu
        var = jnp.mean(hc * hc, axis=1, keepdims=True)
        o_ref[...] = hc * lax.rsqrt(var + EPS) * gm_ref[...] + bt_ref[...]

    return pl.pallas_call(
        body,
        grid=(n_rows // blk,),
        in_specs=[
            pl.BlockSpec((blk, d), lambda i: (i, 0)),
            pl.BlockSpec((t_len, d), lambda i: (0, 0)),
            pl.BlockSpec((1, d), lambda i: (0, 0)),
            pl.BlockSpec((1, d), lambda i: (0, 0)),
        ],
        out_specs=pl.BlockSpec((blk, d), lambda i: (i, 0)),
        out_shape=jax.ShapeDtypeStruct((n_rows, d), jnp.float32),
    )(gathered, pos_table, gamma2, beta2)


def kernel(input_ids, tok_table, pos_table, gamma, beta):
    b, t = input_ids.shape
    v, d = tok_table.shape
    n_rows = b * t

    ids = input_ids.reshape(-1).astype(jnp.int32)
    gathered = _sc_gather(tok_table, ids, n_rows, d)
    out = _tc_add_ln(
        gathered,
        pos_table,
        gamma.reshape(1, d),
        beta.reshape(1, d),
        n_rows,
        t,
        d,
        blk=1024,
    )
    return out.reshape(b, t, d)


# speedup vs baseline: 3.4926x; 1.0420x over previous
import functools

import jax
import jax.numpy as jnp
from jax import lax
from jax.experimental import pallas as pl
from jax.experimental.pallas import tpu as pltpu
from jax.experimental.pallas import tpu_sc as plsc

EPS = 1e-12


def _sc_gather(tok_table, ids, n_rows, d):
    info = plsc.get_sparse_core_info()
    nw = info.num_cores * info.num_subcores
    rows_per_w = n_rows // nw
    chunk = 64

    mesh = plsc.VectorSubcoreMesh(core_axis_name="c", subcore_axis_name="s")

    @functools.partial(
        pl.kernel,
        mesh=mesh,
        out_type=jax.ShapeDtypeStruct((n_rows, d), jnp.float32),
        scratch_types=[
            pltpu.VMEM((rows_per_w,), jnp.int32),
            pltpu.VMEM((chunk, d), jnp.float32),
            pltpu.VMEM((chunk, d), jnp.float32),
            pltpu.SemaphoreType.DMA,
            pltpu.SemaphoreType.DMA,
        ],
    )
    def gather_kernel(table_hbm, idx_hbm, out_hbm, idx_v, r0, r1, s0, s1):
        wid = lax.axis_index("s") * info.num_cores + lax.axis_index("c")
        base = wid * rows_per_w

        pltpu.sync_copy(idx_hbm.at[pl.ds(base, rows_per_w)], idx_v)
        pltpu.async_copy(table_hbm.at[idx_v.at[pl.ds(0, chunk)]], r0, s0)

        @pl.loop(0, rows_per_w, step=2 * chunk)
        def _(c):
            for buf, sem, other_buf, other_sem, off in (
                (r0, s0, r1, s1, chunk),
                (r1, s1, r0, s0, 2 * chunk),
            ):
                nxt = c + off

                @pl.when(nxt < rows_per_w)
                def _():
                    pltpu.async_copy(
                        table_hbm.at[idx_v.at[pl.ds(nxt, chunk)]],
                        other_buf, other_sem)

                pltpu.make_async_copy(table_hbm.at[pl.ds(0, chunk)], buf,
                                      sem).wait()
                pltpu.sync_copy(
                    buf, out_hbm.at[pl.ds(base + nxt - chunk, chunk)])

    return gather_kernel(tok_table, ids)


def _tc_add_ln(gathered, pos_table, gamma2, beta2, n_rows, t_len, d, blk):
    pos_blocks = t_len // blk

    def body(g_ref, p_ref, gm_ref, bt_ref, o_ref):
        i = pl.program_id(0)
        h = g_ref[...] + p_ref[pl.ds((i % pos_blocks) * blk, blk), :]
        mu = jnp.mean(h, axis=1, keepdims=True)
        hc = h - mu
        var = jnp.mean(hc * hc, axis=1, keepdims=True)
        o_ref[...] = hc * lax.rsqrt(var + EPS) * gm_ref[...] + bt_ref[...]

    return pl.pallas_call(
        body,
        grid=(n_rows // blk,),
        in_specs=[
            pl.BlockSpec((blk, d), lambda i: (i, 0)),
            pl.BlockSpec((t_len, d), lambda i: (0, 0)),
            pl.BlockSpec((1, d), lambda i: (0, 0)),
            pl.BlockSpec((1, d), lambda i: (0, 0)),
        ],
        out_specs=pl.BlockSpec((blk, d), lambda i: (i, 0)),
        out_shape=jax.ShapeDtypeStruct((n_rows, d), jnp.float32),
    )(gathered, pos_table, gamma2, beta2)


def kernel(input_ids, tok_table, pos_table, gamma, beta):
    b, t = input_ids.shape
    v, d = tok_table.shape
    n_rows = b * t

    ids = input_ids.reshape(-1).astype(jnp.int32)
    gathered = _sc_gather(tok_table, ids, n_rows, d)
    out = _tc_add_ln(
        gathered,
        pos_table,
        gamma.reshape(1, d),
        beta.reshape(1, d),
        n_rows,
        t,
        d,
        blk=1024,
    )
    return out.reshape(b, t, d)
